# baseline (speedup 1.0000x reference)
import functools

import jax
import jax.numpy as jnp
from jax import lax
from jax.experimental import pallas as pl
from jax.experimental.pallas import tpu as pltpu
from jax.experimental.pallas import tpu_sc as plsc

_B, _C, _H, _W, _N = 8, 64, 384, 384, 128
_HW = _H * _W
_CHW = _C * _HW
_LANES = 16
_TILES = 32
_PPT = (_B * _N) // _TILES
_CHUNKS = _PPT // _LANES


_RING = 6


def _sc_body(pred_hbm, pos_hbm, gt_hbm, out_s, out_c,
             posm, gtv, pbuf, accv, cntv, gt_sem, *sems):
    cid = lax.axis_index("c")
    sid = lax.axis_index("s")
    wid = cid * 16 + sid
    b = wid // (_TILES // _B)
    iota = lax.iota(jnp.int32, _LANES)

    pltpu.sync_copy(pos_hbm.at[pl.ds(wid * _PPT, _PPT), :], posm)
    gt_cp = pltpu.async_copy(gt_hbm.at[pl.ds(wid * _PPT, _PPT), :], gtv,
                             gt_sem)

    xs, ys, vfs = [], [], []
    for k in range(_CHUNKS):
        rowi = iota + k * _LANES
        x = plsc.load_gather(posm, [rowi, iota * 0])
        y = plsc.load_gather(posm, [rowi, iota * 0 + 1])
        vfs.append(jnp.where(x >= 0, jnp.float32(1.0), jnp.float32(0.0)))
        xs.append(jnp.minimum(jnp.maximum(x, 0), _W - 1))
        ys.append(jnp.minimum(jnp.maximum(y, 0), _H - 1))

    _cache = {}

    def scal(p):
        if p not in _cache:
            k, l = p // _LANES, p % _LANES
            pick = iota == l
            _cache[p] = (jnp.sum(jnp.where(pick, xs[k], 0)),
                         jnp.sum(jnp.where(pick, ys[k], 0)),
                         jnp.sum(jnp.where(pick, vfs[k], 0)))
        return _cache[p]

    def start(p):
        x_s, y_s, _ = scal(p)
        x_t = lax.shift_left(lax.shift_right_logical(x_s, 7), 7)
        return pltpu.async_copy(
            pred_hbm.at[b, :, y_s, pl.ds(pl.multiple_of(x_t, 128), 128)],
            pbuf.at[p % _RING], sems[p % _RING])

    copies = [start(p) for p in range(_RING - 1)]
    gt_cp.wait()
    acc = jnp.zeros((_LANES,), jnp.float32)
    for p in range(_PPT):
        copies[p].wait()
        if p + _RING - 1 < _PPT:
            copies.append(start(p + _RING - 1))
        x_s, _, vf = scal(p)
        colv = jnp.full((_LANES,), x_s & 127, jnp.int32)
        for v in range(_C // _LANES):
            pv = plsc.load_gather(pbuf.at[p % _RING], [iota + v * _LANES, colv])
            gv = gtv[p, pl.ds(v * _LANES, _LANES)]
            acc = acc + jnp.abs(pv - gv) * vf

    accv[...] = acc
    cnt = vfs[0]
    for k in range(1, _CHUNKS):
        cnt = cnt + vfs[k]
    cntv[...] = cnt
    pltpu.sync_copy(accv, out_s.at[wid])
    pltpu.sync_copy(cntv, out_c.at[wid])


_sc_gather_loss = functools.partial(
    pl.kernel,
    mesh=plsc.VectorSubcoreMesh(core_axis_name="c", subcore_axis_name="s"),
    compiler_params=pltpu.CompilerParams(needs_layout_passes=False),
    out_type=[
        jax.ShapeDtypeStruct((_TILES, _LANES), jnp.float32),
        jax.ShapeDtypeStruct((_TILES, _LANES), jnp.float32),
    ],
    scratch_types=[
        pltpu.VMEM((_PPT, 2), jnp.int32),
        pltpu.VMEM((_PPT, _C), jnp.float32),
        pltpu.VMEM((_RING, _C, 128), jnp.float32),
        pltpu.VMEM((_LANES,), jnp.float32),
        pltpu.VMEM((_LANES,), jnp.float32),
    ] + [pltpu.SemaphoreType.DMA] * (_RING + 1),
)(_sc_body)


def _finalize_body(s_ref, c_ref, o_ref):
    total = jnp.sum(s_ref[...])
    cnt = jnp.sum(c_ref[...])
    denom = jnp.maximum(cnt * jnp.float32(_C), jnp.float32(1.0))
    o_ref[0, 0] = jnp.where(cnt > 0, total / denom, jnp.float32(0.0))


_finalize = pl.pallas_call(
    _finalize_body,
    out_shape=jax.ShapeDtypeStruct((1, 1), jnp.float32),
    out_specs=pl.BlockSpec(memory_space=pltpu.SMEM),
)


@jax.jit
def kernel(pred_key, gt_pos, gt_key):
    pos2 = gt_pos.astype(jnp.int32).reshape(_B * _N, 2)
    gt2 = gt_key.reshape(_B * _N, _C)
    sums, cnts = _sc_gather_loss(pred_key, pos2, gt2)
    return _finalize(sums, cnts)[0, 0]


# speedup vs baseline: 1.0272x; 1.0272x over previous
import functools

import jax
import jax.numpy as jnp
from jax import lax
from jax.experimental import pallas as pl
from jax.experimental.pallas import tpu as pltpu
from jax.experimental.pallas import tpu_sc as plsc

_B, _C, _H, _W, _N = 8, 64, 384, 384, 128
_HW = _H * _W
_CHW = _C * _HW
_LANES = 16
_TILES = 32
_PPT = (_B * _N) // _TILES
_CHUNKS = _PPT // _LANES


_RING = 8


def _sc_body(pred_hbm, pos_hbm, gt_hbm, out_s, out_c,
             posm, gtv, pbuf, accv, cntv, gt_sem, *sems):
    cid = lax.axis_index("c")
    sid = lax.axis_index("s")
    wid = cid * 16 + sid
    b = wid // (_TILES // _B)
    iota = lax.iota(jnp.int32, _LANES)

    pltpu.sync_copy(pos_hbm.at[pl.ds(wid * _PPT, _PPT), :], posm)
    gt_cp = pltpu.async_copy(gt_hbm.at[pl.ds(wid * _PPT, _PPT), :], gtv,
                             gt_sem)

    xs, ys, vfs = [], [], []
    for k in range(_CHUNKS):
        rowi = iota + k * _LANES
        x = plsc.load_gather(posm, [rowi, iota * 0])
        y = plsc.load_gather(posm, [rowi, iota * 0 + 1])
        vfs.append(jnp.where(x >= 0, jnp.float32(1.0), jnp.float32(0.0)))
        xs.append(jnp.minimum(jnp.maximum(x, 0), _W - 1))
        ys.append(jnp.minimum(jnp.maximum(y, 0), _H - 1))

    _cache = {}

    def scal(p):
        if p not in _cache:
            k, l = p // _LANES, p % _LANES
            pick = iota == l
            _cache[p] = (jnp.sum(jnp.where(pick, xs[k], 0)),
                         jnp.sum(jnp.where(pick, ys[k], 0)),
                         jnp.sum(jnp.where(pick, vfs[k], 0)))
        return _cache[p]

    def start(p):
        x_s, y_s, _ = scal(p)
        x_t = lax.shift_left(lax.shift_right_logical(x_s, 7), 7)
        return pltpu.async_copy(
            pred_hbm.at[b, :, y_s, pl.ds(pl.multiple_of(x_t, 128), 128)],
            pbuf.at[p % _RING], sems[p % _RING])

    copies = [start(p) for p in range(_RING - 1)]
    gt_cp.wait()
    acc = jnp.zeros((_LANES,), jnp.float32)
    for p in range(_PPT):
        copies[p].wait()
        if p + _RING - 1 < _PPT:
            copies.append(start(p + _RING - 1))
        x_s, _, vf = scal(p)
        colv = jnp.full((_LANES,), x_s & 127, jnp.int32)
        for v in range(_C // _LANES):
            pv = plsc.load_gather(pbuf.at[p % _RING], [iota + v * _LANES, colv])
            gv = gtv[p, pl.ds(v * _LANES, _LANES)]
            acc = acc + jnp.abs(pv - gv) * vf

    accv[...] = acc
    cnt = vfs[0]
    for k in range(1, _CHUNKS):
        cnt = cnt + vfs[k]
    cntv[...] = cnt
    pltpu.sync_copy(accv, out_s.at[wid])
    pltpu.sync_copy(cntv, out_c.at[wid])


_sc_gather_loss = functools.partial(
    pl.kernel,
    mesh=plsc.VectorSubcoreMesh(core_axis_name="c", subcore_axis_name="s"),
    compiler_params=pltpu.CompilerParams(needs_layout_passes=False),
    out_type=[
        jax.ShapeDtypeStruct((_TILES, _LANES), jnp.float32),
        jax.ShapeDtypeStruct((_TILES, _LANES), jnp.float32),
    ],
    scratch_types=[
        pltpu.VMEM((_PPT, 2), jnp.int32),
        pltpu.VMEM((_PPT, _C), jnp.float32),
        pltpu.VMEM((_RING, _C, 128), jnp.float32),
        pltpu.VMEM((_LANES,), jnp.float32),
        pltpu.VMEM((_LANES,), jnp.float32),
    ] + [pltpu.SemaphoreType.DMA] * (_RING + 1),
)(_sc_body)


def _finalize_body(s_ref, c_ref, o_ref):
    total = jnp.sum(s_ref[...])
    cnt = jnp.sum(c_ref[...])
    denom = jnp.maximum(cnt * jnp.float32(_C), jnp.float32(1.0))
    o_ref[0, 0] = jnp.where(cnt > 0, total / denom, jnp.float32(0.0))


_finalize = pl.pallas_call(
    _finalize_body,
    out_shape=jax.ShapeDtypeStruct((1, 1), jnp.float32),
    out_specs=pl.BlockSpec(memory_space=pltpu.SMEM),
)


@jax.jit
def kernel(pred_key, gt_pos, gt_key):
    pos2 = gt_pos.astype(jnp.int32).reshape(_B * _N, 2)
    gt2 = gt_key.reshape(_B * _N, _C)
    sums, cnts = _sc_gather_loss(pred_key, pos2, gt2)
    return _finalize(sums, cnts)[0, 0]


# speedup vs baseline: 1.0352x; 1.0078x over previous
import functools

import jax
import jax.numpy as jnp
from jax import lax
from jax.experimental import pallas as pl
from jax.experimental.pallas import tpu as pltpu
from jax.experimental.pallas import tpu_sc as plsc

_B, _C, _H, _W, _N = 8, 64, 384, 384, 128
_HW = _H * _W
_CHW = _C * _HW
_LANES = 16
_TILES = 32
_PPT = (_B * _N) // _TILES
_CHUNKS = _PPT // _LANES


_RING = 12


def _sc_body(pred_hbm, pos_hbm, gt_hbm, out_s, out_c,
             posm, gtv, pbuf, accv, cntv, gt_sem, *sems):
    cid = lax.axis_index("c")
    sid = lax.axis_index("s")
    wid = cid * 16 + sid
    b = wid // (_TILES // _B)
    iota = lax.iota(jnp.int32, _LANES)

    pltpu.sync_copy(pos_hbm.at[pl.ds(wid * _PPT, _PPT), :], posm)
    gt_cp = pltpu.async_copy(gt_hbm.at[pl.ds(wid * _PPT, _PPT), :], gtv,
                             gt_sem)

    xs, ys, vfs = [], [], []
    for k in range(_CHUNKS):
        rowi = iota + k * _LANES
        x = plsc.load_gather(posm, [rowi, iota * 0])
        y = plsc.load_gather(posm, [rowi, iota * 0 + 1])
        vfs.append(jnp.where(x >= 0, jnp.float32(1.0), jnp.float32(0.0)))
        xs.append(jnp.minimum(jnp.maximum(x, 0), _W - 1))
        ys.append(jnp.minimum(jnp.maximum(y, 0), _H - 1))

    _cache = {}

    def scal(p):
        if p not in _cache:
            k, l = p // _LANES, p % _LANES
            pick = iota == l
            _cache[p] = (jnp.sum(jnp.where(pick, xs[k], 0)),
                         jnp.sum(jnp.where(pick, ys[k], 0)),
                         jnp.sum(jnp.where(pick, vfs[k], 0)))
        return _cache[p]

    def start(p):
        x_s, y_s, _ = scal(p)
        x_t = lax.shift_left(lax.shift_right_logical(x_s, 7), 7)
        return pltpu.async_copy(
            pred_hbm.at[b, :, y_s, pl.ds(pl.multiple_of(x_t, 128), 128)],
            pbuf.at[p % _RING], sems[p % _RING])

    copies = [start(p) for p in range(_RING - 1)]
    gt_cp.wait()
    acc = jnp.zeros((_LANES,), jnp.float32)
    for p in range(_PPT):
        copies[p].wait()
        if p + _RING - 1 < _PPT:
            copies.append(start(p + _RING - 1))
        x_s, _, vf = scal(p)
        colv = jnp.full((_LANES,), x_s & 127, jnp.int32)
        for v in range(_C // _LANES):
            pv = plsc.load_gather(pbuf.at[p % _RING], [iota + v * _LANES, colv])
            gv = gtv[p, pl.ds(v * _LANES, _LANES)]
            acc = acc + jnp.abs(pv - gv) * vf

    accv[...] = acc
    cnt = vfs[0]
    for k in range(1, _CHUNKS):
        cnt = cnt + vfs[k]
    cntv[...] = cnt
    pltpu.sync_copy(accv, out_s.at[wid])
    pltpu.sync_copy(cntv, out_c.at[wid])


_sc_gather_loss = functools.partial(
    pl.kernel,
    mesh=plsc.VectorSubcoreMesh(core_axis_name="c", subcore_axis_name="s"),
    compiler_params=pltpu.CompilerParams(needs_layout_passes=False),
    out_type=[
        jax.ShapeDtypeStruct((_TILES, _LANES), jnp.float32),
        jax.ShapeDtypeStruct((_TILES, _LANES), jnp.float32),
    ],
    scratch_types=[
        pltpu.VMEM((_PPT, 2), jnp.int32),
        pltpu.VMEM((_PPT, _C), jnp.float32),
        pltpu.VMEM((_RING, _C, 128), jnp.float32),
        pltpu.VMEM((_LANES,), jnp.float32),
        pltpu.VMEM((_LANES,), jnp.float32),
    ] + [pltpu.SemaphoreType.DMA] * (_RING + 1),
)(_sc_body)


def _finalize_body(s_ref, c_ref, o_ref):
    total = jnp.sum(s_ref[...])
    cnt = jnp.sum(c_ref[...])
    denom = jnp.maximum(cnt * jnp.float32(_C), jnp.float32(1.0))
    o_ref[0, 0] = jnp.where(cnt > 0, total / denom, jnp.float32(0.0))


_finalize = pl.pallas_call(
    _finalize_body,
    out_shape=jax.ShapeDtypeStruct((1, 1), jnp.float32),
    out_specs=pl.BlockSpec(memory_space=pltpu.SMEM),
)


@jax.jit
def kernel(pred_key, gt_pos, gt_key):
    pos2 = gt_pos.astype(jnp.int32).reshape(_B * _N, 2)
    gt2 = gt_key.reshape(_B * _N, _C)
    sums, cnts = _sc_gather_loss(pred_key, pos2, gt2)
    return _finalize(sums, cnts)[0, 0]
